# baseline (speedup 1.0000x reference)
import functools

import jax
import jax.numpy as jnp
from jax import lax
from jax.experimental import pallas as pl
from jax.experimental.pallas import tpu as pltpu
from jax.experimental.pallas import tpu_sc as plsc

_BLOCK_ROWS = 2048
_BUFFER_COUNT = 4


def _log_inner(idx, x_blk, o_blk):
    del idx
    o_blk[...] = jnp.log(x_blk[...])


def _log_outer(x_hbm, o_hbm):
    n_rows, n_cols = x_hbm.shape
    in_spec = pl.BlockSpec(
        (_BLOCK_ROWS, n_cols),
        lambda i: (i, 0),
        pipeline_mode=pl.Buffered(buffer_count=_BUFFER_COUNT),
    )
    out_spec = pl.BlockSpec((_BLOCK_ROWS, n_cols), lambda i: (i, 0))
    pipe = pltpu.emit_pipeline(
        _log_inner,
        grid=(n_rows // _BLOCK_ROWS,),
        in_specs=[in_spec],
        out_specs=[out_spec],
        _explicit_indices=True,
    )
    pipe(x_hbm, o_hbm)


def _tc_log(x):
    n_rows, n_cols = x.shape
    return pl.pallas_call(
        _log_outer,
        in_specs=[pl.BlockSpec(memory_space=pl.ANY)],
        out_specs=pl.BlockSpec(memory_space=pl.ANY),
        out_shape=jax.ShapeDtypeStruct((n_rows, n_cols), x.dtype),
    )(x)


_SC_MESH = plsc.VectorSubcoreMesh(core_axis_name="c", subcore_axis_name="s")


@functools.partial(
    pl.kernel,
    mesh=_SC_MESH,
    scratch_types=[
        pltpu.VMEM((1024,), jnp.float32),
        pltpu.VMEM((16,), jnp.float32),
    ],
)
def _sc_patch(y_hbm, row_v, elt_v):
    cid = lax.axis_index("c")
    sid = lax.axis_index("s")

    @pl.when((cid == 0) & (sid == 0))
    def _():
        for i in range(1024 // 16):
            row_v[pl.ds(i * 16, 16)] = jnp.full((16,), 1.0, jnp.float32)
        pltpu.sync_copy(row_v, y_hbm.at[2])

        pltpu.sync_copy(y_hbm.at[1, pl.ds(0, 16)], elt_v)
        lane = lax.broadcasted_iota(jnp.int32, (16,), 0)
        elt_v[...] = jnp.where(lane == 1, jnp.float32(5.0), elt_v[...])
        pltpu.sync_copy(elt_v, y_hbm.at[1, pl.ds(0, 16)])


def kernel(x):
    y_ref = jax.new_ref(_tc_log(x))
    _sc_patch(y_ref)
    return jax.freeze(y_ref)


# speedup vs baseline: 1.3374x; 1.3374x over previous
import jax
import jax.numpy as jnp
from jax.experimental import pallas as pl
from jax.experimental.pallas import tpu as pltpu

_EDGE_ROWS = 2048
_EDGE_BLOCK = 512
_MID_BLOCK = 2048
_BUFFER_COUNT = 4


def _patch_first_rows(o_blk):
    blk = o_blk[0:8, :]
    rows = jax.lax.broadcasted_iota(jnp.int32, blk.shape, 0)
    cols = jax.lax.broadcasted_iota(jnp.int32, blk.shape, 1)
    blk = jnp.where(rows == 2, jnp.float32(1.0), blk)
    blk = jnp.where((rows == 1) & (cols == 1), jnp.float32(5.0), blk)
    o_blk[0:8, :] = blk


def _make_pipe(body, n_rows, n_cols, block_rows):
    in_spec = pl.BlockSpec(
        (block_rows, n_cols),
        lambda i: (i, 0),
        pipeline_mode=pl.Buffered(buffer_count=_BUFFER_COUNT),
    )
    out_spec = pl.BlockSpec((block_rows, n_cols), lambda i: (i, 0))
    return pltpu.emit_pipeline(
        body,
        grid=(n_rows // block_rows,),
        in_specs=[in_spec],
        out_specs=[out_spec],
        _explicit_indices=True,
    )


def _log_patch_body(idx, x_blk, o_blk):
    (i,) = idx
    o_blk[...] = jnp.log(x_blk[...])

    @pl.when(i == 0)
    def _():
        _patch_first_rows(o_blk)


def _log_body(idx, x_blk, o_blk):
    del idx
    o_blk[...] = jnp.log(x_blk[...])


def _outer(x_hbm, o_hbm):
    n_rows, n_cols = x_hbm.shape
    mid_rows = n_rows - 2 * _EDGE_ROWS

    head = _make_pipe(_log_patch_body, _EDGE_ROWS, n_cols, _EDGE_BLOCK)
    mid = _make_pipe(_log_body, mid_rows, n_cols, _MID_BLOCK)
    tail = _make_pipe(_log_body, _EDGE_ROWS, n_cols, _EDGE_BLOCK)

    head(x_hbm.at[pl.ds(0, _EDGE_ROWS), :], o_hbm.at[pl.ds(0, _EDGE_ROWS), :])
    mid(
        x_hbm.at[pl.ds(_EDGE_ROWS, mid_rows), :],
        o_hbm.at[pl.ds(_EDGE_ROWS, mid_rows), :],
    )
    tail(
        x_hbm.at[pl.ds(n_rows - _EDGE_ROWS, _EDGE_ROWS), :],
        o_hbm.at[pl.ds(n_rows - _EDGE_ROWS, _EDGE_ROWS), :],
    )


def kernel(x):
    n_rows, n_cols = x.shape
    return pl.pallas_call(
        _outer,
        in_specs=[pl.BlockSpec(memory_space=pl.ANY)],
        out_specs=pl.BlockSpec(memory_space=pl.ANY),
        out_shape=jax.ShapeDtypeStruct((n_rows, n_cols), x.dtype),
    )(x)


# speedup vs baseline: 1.4210x; 1.0625x over previous
import jax
import jax.numpy as jnp
from jax.experimental import pallas as pl
from jax.experimental.pallas import tpu as pltpu

_BLOCK_ROWS = 2048
_BUFFER_COUNT = 4


def _patch_first_rows(o_blk):
    blk = o_blk[0:8, :]
    rows = jax.lax.broadcasted_iota(jnp.int32, blk.shape, 0)
    cols = jax.lax.broadcasted_iota(jnp.int32, blk.shape, 1)
    blk = jnp.where(rows == 2, jnp.float32(1.0), blk)
    blk = jnp.where((rows == 1) & (cols == 1), jnp.float32(5.0), blk)
    o_blk[0:8, :] = blk


def _outer(x_hbm, o_hbm):
    n_rows, n_cols = x_hbm.shape

    def _inner(idx, x_blk, o_blk):
        (i,) = idx
        o_blk[...] = jnp.log(x_blk[...])

        @pl.when(i == 0)
        def _():
            _patch_first_rows(o_blk)

    in_spec = pl.BlockSpec(
        (_BLOCK_ROWS, n_cols),
        lambda i: (i, 0),
        pipeline_mode=pl.Buffered(buffer_count=_BUFFER_COUNT),
    )
    out_spec = pl.BlockSpec((_BLOCK_ROWS, n_cols), lambda i: (i, 0))
    pipe = pltpu.emit_pipeline(
        _inner,
        grid=(n_rows // _BLOCK_ROWS,),
        in_specs=[in_spec],
        out_specs=[out_spec],
        _explicit_indices=True,
    )
    pipe(x_hbm, o_hbm)


def kernel(x):
    n_rows, n_cols = x.shape
    return pl.pallas_call(
        _outer,
        in_specs=[pl.BlockSpec(memory_space=pl.ANY)],
        out_specs=pl.BlockSpec(memory_space=pl.ANY),
        out_shape=jax.ShapeDtypeStruct((n_rows, n_cols), x.dtype),
    )(x)
